# baseline (speedup 1.0000x reference)
import functools

import jax
import jax.numpy as jnp
from jax import lax
from jax.experimental import pallas as pl
from jax.experimental.pallas import tpu as pltpu
from jax.experimental.pallas import tpu_sc as plsc

B = 128
N = 32768
K = 64
L = 16
NV = N // L
NC = 2
NS = 16
NW = NC * NS
ROWS_PER_W = B // NW
CMAX = 4096
NEG = float("-inf")
BIG = 1 << 30


def _splat_f(x):
    return jnp.full((L,), x, dtype=jnp.float32)


def _splat_i(x):
    return jnp.full((L,), x, dtype=jnp.int32)


SU = 4


def _select_topk(val_load, val_kill, idx_of, ngroups, ovbuf, oibuf, lane0, iota):
    def k_body(k, _):
        def scan(jg, carry):
            lmax, lpos = carry
            for u in range(SU):
                j = jg * SU + u
                x = val_load(j)
                gt = x > lmax
                lmax = jnp.maximum(lmax, x)
                lpos = jnp.where(gt, _splat_i(j * L) + iota, lpos)
            return (lmax, lpos)

        lmax, lpos = lax.fori_loop(
            0, ngroups, scan, (_splat_f(NEG), _splat_i(0)))
        m = jnp.max(lmax)
        msp = _splat_f(m)
        cand = jnp.where(lmax == msp, lpos, BIG)
        found = jnp.min(cand)
        fsp = _splat_i(found)
        plsc.store_scatter(ovbuf, [_splat_i(k)], msp, mask=lane0)
        plsc.store_scatter(oibuf, [_splat_i(k)], idx_of(fsp), mask=lane0)
        val_kill(fsp)
        return 0

    lax.fori_loop(0, K, k_body, 0)


def _topk_body(acc_hbm, outv_hbm, outi_hbm,
               rowbuf0, rowbuf1, cval, cidx, gmaxbuf, ovbuf, oibuf,
               sem0, sem1):
    wid = lax.axis_index("s") * NC + lax.axis_index("c")
    base_row = wid * ROWS_PER_W
    sems = (sem0, sem1)
    iota = lax.broadcasted_iota(jnp.int32, (L,), 0)
    lane0 = iota == 0
    neg16 = _splat_f(NEG)

    bufs = (rowbuf0, rowbuf1)
    handles = [None, None]
    handles[0] = pltpu.async_copy(acc_hbm.at[base_row], bufs[0], sems[0])
    for r in range(ROWS_PER_W):
        cur = r % 2
        nxt = (r + 1) % 2
        if r + 1 < ROWS_PER_W:
            handles[nxt] = pltpu.async_copy(
                acc_hbm.at[base_row + (r + 1)], bufs[nxt], sems[nxt])
        handles[cur].wait()
        row = bufs[cur]

        G = 8
        NG = NV // G
        GPB = NG // 4

        accs = [neg16, neg16, neg16, neg16]
        for b in range(4):
            def p1(g, acc, b=b):
                base = (b * GPB + g) * (G * L)
                x0 = row[pl.ds(base + 0 * L, L)]
                x1 = row[pl.ds(base + 1 * L, L)]
                x2 = row[pl.ds(base + 2 * L, L)]
                x3 = row[pl.ds(base + 3 * L, L)]
                x4 = row[pl.ds(base + 4 * L, L)]
                x5 = row[pl.ds(base + 5 * L, L)]
                x6 = row[pl.ds(base + 6 * L, L)]
                x7 = row[pl.ds(base + 7 * L, L)]
                m01 = jnp.maximum(x0, x1)
                m23 = jnp.maximum(x2, x3)
                m45 = jnp.maximum(x4, x5)
                m67 = jnp.maximum(x6, x7)
                mx = jnp.maximum(jnp.maximum(m01, m23),
                                 jnp.maximum(m45, m67))
                gmaxbuf[pl.ds((b * GPB + g) * L, L)] = mx
                return jnp.maximum(acc, mx)

            accs[b] = lax.fori_loop(0, GPB, p1, accs[b])

        a0, a1, a2, a3 = accs
        thr = jnp.min(jnp.minimum(jnp.minimum(a0, a1), jnp.minimum(a2, a3)))
        thr_s = _splat_f(thr)

        lim_s = _splat_i(CMAX + L)

        def p2(g, cntv):
            base = g * (G * L)
            has = jnp.any(gmaxbuf[pl.ds(g * L, L)] >= thr_s)

            def taken(cntv):
                for k in range(G):
                    xk = row[pl.ds(base + k * L, L)]
                    msk = xk >= thr_s
                    pfx = plsc.cumsum(msk.astype(jnp.int32))
                    tgt = cntv + pfx - 1
                    ok = msk & (tgt < lim_s)
                    plsc.store_scatter(cval, [tgt], xk, mask=ok)
                    plsc.store_scatter(
                        cidx, [tgt], iota + (base + k * L), mask=ok)
                    cntv = cntv + plsc.all_reduce_population_count(msk)
                return cntv

            return lax.cond(has, taken, lambda z: z, cntv)

        cntv = lax.fori_loop(0, NG, p2, _splat_i(0))
        cnt = jnp.max(cntv)
        padbase = jnp.minimum(cnt, CMAX)
        for u in range(SU):
            cval[pl.ds(padbase + u * L, L)] = neg16

        def normal(_):
            _select_topk(
                val_load=lambda j: cval[pl.ds(j * L, L)],
                val_kill=lambda fsp: plsc.store_scatter(
                    cval, [fsp], neg16, mask=lane0),
                idx_of=lambda fsp: plsc.load_gather(cidx, [fsp]),
                ngroups=(cnt + SU * L - 1) // (SU * L),
                ovbuf=ovbuf, oibuf=oibuf, lane0=lane0, iota=iota)
            return 0

        def fallback(_):
            _select_topk(
                val_load=lambda j: row[pl.ds(j * L, L)],
                val_kill=lambda fsp: plsc.store_scatter(
                    row, [fsp], neg16, mask=lane0),
                idx_of=lambda fsp: fsp,
                ngroups=NV // SU,
                ovbuf=ovbuf, oibuf=oibuf, lane0=lane0, iota=iota)
            return 0

        lax.cond(cnt <= CMAX, normal, fallback, 0)

        pltpu.sync_copy(ovbuf, outv_hbm.at[base_row + r])
        pltpu.sync_copy(oibuf, outi_hbm.at[base_row + r])


@functools.lru_cache(maxsize=1)
def _topk_call():
    return functools.partial(
        pl.kernel,
        out_type=[
            jax.ShapeDtypeStruct((B, K), jnp.float32),
            jax.ShapeDtypeStruct((B, K), jnp.int32),
        ],
        mesh=plsc.VectorSubcoreMesh(core_axis_name="c", subcore_axis_name="s"),
        compiler_params=pltpu.CompilerParams(needs_layout_passes=False),
        scratch_types=[
            pltpu.VMEM((N,), jnp.float32),
            pltpu.VMEM((N,), jnp.float32),
            pltpu.VMEM((CMAX + SU * L,), jnp.float32),
            pltpu.VMEM((CMAX + SU * L,), jnp.int32),
            pltpu.VMEM((N // 8,), jnp.float32),
            pltpu.VMEM((K,), jnp.float32),
            pltpu.VMEM((K,), jnp.int32),
            pltpu.SemaphoreType.DMA,
            pltpu.SemaphoreType.DMA,
        ],
    )(_topk_body)


def kernel(accumulated):
    topk_vals, topk_idx = _topk_call()(accumulated)
    return (topk_vals, topk_idx, accumulated)
